# baseline (speedup 1.0000x reference)
import functools

import jax
import jax.numpy as jnp
from jax import lax
from jax.experimental import pallas as pl
from jax.experimental.pallas import tpu as pltpu
from jax.experimental.pallas import tpu_sc as plsc

N = 10000
E = 160000
NC = 2
NS = 16
NW = NC * NS
EPW = E // NW
CHUNK = 96
EPT = 5184
GROUP = 3
NCHUNK = EPT // CHUNK
NGROUP = NCHUNK // GROUP
NP = 10240
SLAB = NP // NS
EPW_P = 5008
PAD = NW * EPW_P - E
RED = 128
BLK = 1000

_MESH = plsc.VectorSubcoreMesh(core_axis_name="c", subcore_axis_name="s",
                               num_cores=NC, num_subcores=NS)
_SC_PARAMS = pltpu.CompilerParams(needs_layout_passes=False,
                                  use_tc_tiling_on_sc=False)


@functools.partial(
    pl.kernel,
    out_type=jax.ShapeDtypeStruct((NW, NP, 1), jnp.float32),
    mesh=_MESH,
    compiler_params=_SC_PARAMS,
    scratch_types=[
        pltpu.VMEM((EPW_P,), jnp.int32),
        pltpu.VMEM((NP, 1), jnp.float32),
    ],
)
def _degree_kernel(col_hbm, zeros_hbm, hist_out, colv, hist):
    cid = lax.axis_index("c")
    sid = lax.axis_index("s")
    wid = cid * NS + sid
    pltpu.sync_copy(col_hbm.at[wid], colv)
    pltpu.sync_copy(zeros_hbm, hist)
    ones = jnp.full((16,), 1.0, jnp.float32)
    zcol = jnp.zeros((16,), jnp.int32)

    def body(j, carry):
        idx = colv[pl.ds(j * 16, 16)]
        plsc.addupdate_scatter(hist, [idx, zcol], ones)
        return carry

    lax.fori_loop(0, EPW_P // 16, body, 0)
    pltpu.sync_copy(hist, hist_out.at[wid])


def _make_propagate(ntab, pipelined):
    nbuf = 2 * GROUP if pipelined else GROUP
    scratch = (
        [pltpu.VMEM((NCHUNK, CHUNK), jnp.int32) for _ in range(2)]
        + [pltpu.VMEM((CHUNK, 64), jnp.float32) for _ in range(nbuf)]
        + [pltpu.VMEM_SHARED((NP, 64), jnp.float32),
           pltpu.SemaphoreType.DMA, pltpu.SemaphoreType.DMA]
    )
    out_type = [jax.ShapeDtypeStruct((NC, NP, 64), jnp.float32)
                for _ in range(ntab)]

    @functools.partial(pl.kernel, out_type=out_type, mesh=_MESH,
                       compiler_params=_SC_PARAMS, scratch_types=scratch)
    def prop(*refs):
        tabs = refs[:ntab]
        row_hbm, col_hbm, zeros_hbm = refs[ntab:ntab + 3]
        outs = refs[ntab + 3:2 * ntab + 3]
        rowbuf, colbuf = refs[2 * ntab + 3:2 * ntab + 5]
        allbufs = refs[2 * ntab + 5:2 * ntab + 5 + nbuf]
        bufs = (allbufs[:GROUP], allbufs[GROUP:])
        acc = refs[-3]
        sems = (refs[-2], refs[-1])
        cid = lax.axis_index("c")
        sid = lax.axis_index("s")
        wid = cid * NS + sid
        pltpu.sync_copy(row_hbm.at[wid], rowbuf)
        pltpu.sync_copy(col_hbm.at[wid], colbuf)
        slab = pl.ds(sid * SLAB, SLAB)

        def fire(t, g, s):
            return [pltpu.async_copy(
                tabs[t].at[rowbuf.at[g * GROUP + b]], bufs[s][b], sems[s])
                for b in range(GROUP)]

        def scat(g, s):
            for b in range(GROUP):
                pltpu.sync_copy(bufs[s][b], acc.at[colbuf.at[g * GROUP + b]],
                                add=True)

        for t in range(ntab):
            pltpu.sync_copy(zeros_hbm, acc.at[slab])
            plsc.subcore_barrier()
            if pipelined:
                def body(it, carry, t=t):
                    g0 = it * 2
                    da = fire(t, g0, 0)
                    db = fire(t, g0 + 1, 1)
                    for d in da:
                        d.wait()
                    scat(g0, 0)
                    for d in db:
                        d.wait()
                    scat(g0 + 1, 1)
                    return carry

                lax.fori_loop(0, NGROUP // 2, body, 0)
            else:
                def body(g, carry, t=t):
                    descs = fire(t, g, 0)
                    for d in descs:
                        d.wait()
                    scat(g, 0)
                    return carry

                lax.fori_loop(0, NGROUP, body, 0)
            plsc.subcore_barrier()
            pltpu.sync_copy(acc.at[slab], outs[t].at[cid, slab])

    return prop


_prop_l1 = _make_propagate(4, pipelined=True)
_prop_l2 = _make_propagate(1, pipelined=False)


def _prep(x, hists):
    def body(x_ref, h_ref, xs0_ref, xs1_ref, xs2_ref, xs3_ref, dinv_ref):
        deg = jnp.sum(h_ref[...], axis=0) + 1.0
        dinv = lax.rsqrt(deg)
        xs = x_ref[...] * dinv
        xs0_ref[...] = xs[:, 0:64]
        xs1_ref[...] = xs[:, 64:128]
        xs2_ref[...] = xs[:, 128:192]
        xs3_ref[...] = xs[:, 192:256]
        dinv_ref[...] = dinv

    return pl.pallas_call(
        body,
        grid=(N // BLK,),
        in_specs=[
            pl.BlockSpec((BLK, 256), lambda i: (i, 0)),
            pl.BlockSpec((NW, BLK, 1), lambda i: (0, i, 0)),
        ],
        out_specs=[pl.BlockSpec((BLK, 64), lambda i: (i, 0))] * 4
        + [pl.BlockSpec((BLK, 1), lambda i: (i, 0))],
        out_shape=[jax.ShapeDtypeStruct((N, 64), jnp.float32)] * 4
        + [jax.ShapeDtypeStruct((N, 1), jnp.float32)],
    )(x, hists)


def _mm(ps, xss, dinv, W1, b1, W2):
    def body(p0_ref, p1_ref, p2_ref, p3_ref, xs0_ref, xs1_ref, xs2_ref,
             xs3_ref, dinv_ref, W1_ref, b1_ref, W2_ref, z_ref):
        dv = dinv_ref[...]
        p_refs = (p0_ref, p1_ref, p2_ref, p3_ref)
        xs_refs = (xs0_ref, xs1_ref, xs2_ref, xs3_ref)
        h = b1_ref[...]
        for k in range(4):
            u = (p_refs[k][0] + p_refs[k][1] + xs_refs[k][...]) * dv
            h = h + jnp.dot(u, W1_ref[64 * k:64 * (k + 1), :],
                            preferred_element_type=jnp.float32)
        hs = jnp.maximum(h, 0.0) * dv
        z_ref[...] = jnp.dot(hs, W2_ref[...], preferred_element_type=jnp.float32)

    return pl.pallas_call(
        body,
        grid=(N // BLK,),
        in_specs=[pl.BlockSpec((NC, BLK, 64), lambda i: (0, i, 0))] * 4
        + [pl.BlockSpec((BLK, 64), lambda i: (i, 0))] * 4
        + [
            pl.BlockSpec((BLK, 1), lambda i: (i, 0)),
            pl.BlockSpec((256, 512), lambda i: (0, 0)),
            pl.BlockSpec((1, 512), lambda i: (0, 0)),
            pl.BlockSpec((512, 64), lambda i: (0, 0)),
        ],
        out_specs=pl.BlockSpec((BLK, 64), lambda i: (i, 0)),
        out_shape=jax.ShapeDtypeStruct((N, 64), jnp.float32),
    )(*ps, *xss, dinv, W1, b1, W2)


def _final(q, z, dinv, b2):
    def body(q_ref, z_ref, dinv_ref, b2_ref, out_ref):
        agg = q_ref[0] + q_ref[1] + z_ref[...]
        out_ref[...] = agg * dinv_ref[...] + b2_ref[...]

    return pl.pallas_call(
        body,
        grid=(N // BLK,),
        in_specs=[
            pl.BlockSpec((NC, BLK, 64), lambda i: (0, i, 0)),
            pl.BlockSpec((BLK, 64), lambda i: (i, 0)),
            pl.BlockSpec((BLK, 1), lambda i: (i, 0)),
            pl.BlockSpec((1, 64), lambda i: (0, 0)),
        ],
        out_specs=pl.BlockSpec((BLK, 64), lambda i: (i, 0)),
        out_shape=jax.ShapeDtypeStruct((N, 64), jnp.float32),
    )(q, z, dinv, b2)


def kernel(x, edge_index, W1, b1, W2, b2):
    ei = edge_index.astype(jnp.int32)
    npad = NW * (EPT - EPW)
    fill_r = (jnp.arange(npad, dtype=jnp.int32) % N).reshape(NW, EPT - EPW)
    fill_c = (N + jnp.arange(npad, dtype=jnp.int32) % (NP - N)).reshape(
        NW, EPT - EPW)
    row = jnp.concatenate([ei[0].reshape(NW, EPW), fill_r],
                          axis=1).reshape(NW, NCHUNK, CHUNK)
    col = jnp.concatenate([ei[1].reshape(NW, EPW), fill_c],
                          axis=1).reshape(NW, NCHUNK, CHUNK)
    colp = jnp.concatenate(
        [ei[1], jnp.full((PAD,), N, jnp.int32)]).reshape(NW, EPW_P)
    zeros_n = jnp.zeros((NP, 1), jnp.float32)
    z64 = jnp.zeros((SLAB, 64), jnp.float32)
    hists = _degree_kernel(colp, zeros_n)
    *xss, dinv = _prep(x, hists)
    ps = _prop_l1(*xss, row, col, z64)
    z = _mm(ps, xss, dinv, W1, b1[None, :], W2)
    (q,) = _prop_l2(z, row, col, z64)
    out = _final(q, z, dinv, b2[None, :])
    return (out, edge_index)


# speedup vs baseline: 1.0369x; 1.0369x over previous
import functools

import jax
import jax.numpy as jnp
from jax import lax
from jax.experimental import pallas as pl
from jax.experimental.pallas import tpu as pltpu
from jax.experimental.pallas import tpu_sc as plsc

N = 10000
E = 160000
NC = 2
NS = 16
NW = NC * NS
EPW = E // NW
CHUNK = 128
EPT = 5120
GROUP = 4
NCHUNK = EPT // CHUNK
NGROUP = NCHUNK // GROUP
NP = 10240
SLAB = NP // NS
EPW_P = 5008
PAD = NW * EPW_P - E
RED = 128
BLK = 1000

_MESH = plsc.VectorSubcoreMesh(core_axis_name="c", subcore_axis_name="s",
                               num_cores=NC, num_subcores=NS)
_SC_PARAMS = pltpu.CompilerParams(needs_layout_passes=False,
                                  use_tc_tiling_on_sc=False)


@functools.partial(
    pl.kernel,
    out_type=jax.ShapeDtypeStruct((NW, NP, 1), jnp.float32),
    mesh=_MESH,
    compiler_params=_SC_PARAMS,
    scratch_types=[
        pltpu.VMEM((EPW_P,), jnp.int32),
        pltpu.VMEM((NP, 1), jnp.float32),
    ],
)
def _degree_kernel(col_hbm, zeros_hbm, hist_out, colv, hist):
    cid = lax.axis_index("c")
    sid = lax.axis_index("s")
    wid = cid * NS + sid
    pltpu.sync_copy(col_hbm.at[wid], colv)
    pltpu.sync_copy(zeros_hbm, hist)
    ones = jnp.full((16,), 1.0, jnp.float32)
    zcol = jnp.zeros((16,), jnp.int32)

    def body(j, carry):
        idx = colv[pl.ds(j * 16, 16)]
        plsc.addupdate_scatter(hist, [idx, zcol], ones)
        return carry

    lax.fori_loop(0, EPW_P // 16, body, 0)
    pltpu.sync_copy(hist, hist_out.at[wid])


def _make_propagate(ntab, pipelined):
    nbuf = 2 * GROUP if pipelined else GROUP
    scratch = (
        [pltpu.VMEM((NCHUNK, CHUNK), jnp.int32) for _ in range(2)]
        + [pltpu.VMEM((CHUNK, 64), jnp.float32) for _ in range(nbuf)]
        + [pltpu.VMEM_SHARED((NP, 64), jnp.float32),
           pltpu.SemaphoreType.DMA, pltpu.SemaphoreType.DMA]
    )
    out_type = [jax.ShapeDtypeStruct((NC, NP, 64), jnp.float32)
                for _ in range(ntab)]

    @functools.partial(pl.kernel, out_type=out_type, mesh=_MESH,
                       compiler_params=_SC_PARAMS, scratch_types=scratch)
    def prop(*refs):
        tabs = refs[:ntab]
        row_hbm, col_hbm, zeros_hbm = refs[ntab:ntab + 3]
        outs = refs[ntab + 3:2 * ntab + 3]
        rowbuf, colbuf = refs[2 * ntab + 3:2 * ntab + 5]
        allbufs = refs[2 * ntab + 5:2 * ntab + 5 + nbuf]
        bufs = (allbufs[:GROUP], allbufs[GROUP:])
        acc = refs[-3]
        sems = (refs[-2], refs[-1])
        cid = lax.axis_index("c")
        sid = lax.axis_index("s")
        wid = cid * NS + sid
        pltpu.sync_copy(row_hbm.at[wid], rowbuf)
        pltpu.sync_copy(col_hbm.at[wid], colbuf)
        slab = pl.ds(sid * SLAB, SLAB)

        def fire(t, g, s):
            return [pltpu.async_copy(
                tabs[t].at[rowbuf.at[g * GROUP + b]], bufs[s][b], sems[s])
                for b in range(GROUP)]

        def scat(g, s):
            for b in range(GROUP):
                pltpu.sync_copy(bufs[s][b], acc.at[colbuf.at[g * GROUP + b]],
                                add=True)

        for t in range(ntab):
            pltpu.sync_copy(zeros_hbm, acc.at[slab])
            plsc.subcore_barrier()
            if pipelined:
                def body(it, carry, t=t):
                    g0 = it * 2
                    da = fire(t, g0, 0)
                    db = fire(t, g0 + 1, 1)
                    for d in da:
                        d.wait()
                    scat(g0, 0)
                    for d in db:
                        d.wait()
                    scat(g0 + 1, 1)
                    return carry

                lax.fori_loop(0, NGROUP // 2, body, 0)
            else:
                def body(g, carry, t=t):
                    descs = fire(t, g, 0)
                    for d in descs:
                        d.wait()
                    scat(g, 0)
                    return carry

                lax.fori_loop(0, NGROUP, body, 0)
            plsc.subcore_barrier()
            pltpu.sync_copy(acc.at[slab], outs[t].at[cid, slab])

    return prop


_prop_l1 = _make_propagate(4, pipelined=True)
_prop_l2 = _make_propagate(1, pipelined=False)


def _prep(x, hists):
    def body(x_ref, h_ref, xs0_ref, xs1_ref, xs2_ref, xs3_ref, dinv_ref):
        deg = jnp.sum(h_ref[...], axis=0) + 1.0
        dinv = lax.rsqrt(deg)
        xs = x_ref[...] * dinv
        xs0_ref[...] = xs[:, 0:64]
        xs1_ref[...] = xs[:, 64:128]
        xs2_ref[...] = xs[:, 128:192]
        xs3_ref[...] = xs[:, 192:256]
        dinv_ref[...] = dinv

    return pl.pallas_call(
        body,
        grid=(N // BLK,),
        in_specs=[
            pl.BlockSpec((BLK, 256), lambda i: (i, 0)),
            pl.BlockSpec((NW, BLK, 1), lambda i: (0, i, 0)),
        ],
        out_specs=[pl.BlockSpec((BLK, 64), lambda i: (i, 0))] * 4
        + [pl.BlockSpec((BLK, 1), lambda i: (i, 0))],
        out_shape=[jax.ShapeDtypeStruct((N, 64), jnp.float32)] * 4
        + [jax.ShapeDtypeStruct((N, 1), jnp.float32)],
    )(x, hists)


def _mm(ps, xss, dinv, W1, b1, W2):
    def body(p0_ref, p1_ref, p2_ref, p3_ref, xs0_ref, xs1_ref, xs2_ref,
             xs3_ref, dinv_ref, W1_ref, b1_ref, W2_ref, z_ref):
        dv = dinv_ref[...]
        p_refs = (p0_ref, p1_ref, p2_ref, p3_ref)
        xs_refs = (xs0_ref, xs1_ref, xs2_ref, xs3_ref)
        h = b1_ref[...]
        for k in range(4):
            u = (p_refs[k][0] + p_refs[k][1] + xs_refs[k][...]) * dv
            h = h + jnp.dot(u, W1_ref[64 * k:64 * (k + 1), :],
                            preferred_element_type=jnp.float32)
        hs = jnp.maximum(h, 0.0) * dv
        z_ref[...] = jnp.dot(hs, W2_ref[...], preferred_element_type=jnp.float32)

    return pl.pallas_call(
        body,
        grid=(N // BLK,),
        in_specs=[pl.BlockSpec((NC, BLK, 64), lambda i: (0, i, 0))] * 4
        + [pl.BlockSpec((BLK, 64), lambda i: (i, 0))] * 4
        + [
            pl.BlockSpec((BLK, 1), lambda i: (i, 0)),
            pl.BlockSpec((256, 512), lambda i: (0, 0)),
            pl.BlockSpec((1, 512), lambda i: (0, 0)),
            pl.BlockSpec((512, 64), lambda i: (0, 0)),
        ],
        out_specs=pl.BlockSpec((BLK, 64), lambda i: (i, 0)),
        out_shape=jax.ShapeDtypeStruct((N, 64), jnp.float32),
    )(*ps, *xss, dinv, W1, b1, W2)


def _final(q, z, dinv, b2):
    def body(q_ref, z_ref, dinv_ref, b2_ref, out_ref):
        agg = q_ref[0] + q_ref[1] + z_ref[...]
        out_ref[...] = agg * dinv_ref[...] + b2_ref[...]

    return pl.pallas_call(
        body,
        grid=(N // BLK,),
        in_specs=[
            pl.BlockSpec((NC, BLK, 64), lambda i: (0, i, 0)),
            pl.BlockSpec((BLK, 64), lambda i: (i, 0)),
            pl.BlockSpec((BLK, 1), lambda i: (i, 0)),
            pl.BlockSpec((1, 64), lambda i: (0, 0)),
        ],
        out_specs=pl.BlockSpec((BLK, 64), lambda i: (i, 0)),
        out_shape=jax.ShapeDtypeStruct((N, 64), jnp.float32),
    )(q, z, dinv, b2)


def kernel(x, edge_index, W1, b1, W2, b2):
    ei = edge_index.astype(jnp.int32)
    npad = NW * (EPT - EPW)
    fill_r = (jnp.arange(npad, dtype=jnp.int32) % N).reshape(NW, EPT - EPW)
    fill_c = (N + jnp.arange(npad, dtype=jnp.int32) % (NP - N)).reshape(
        NW, EPT - EPW)
    row = jnp.concatenate([ei[0].reshape(NW, EPW), fill_r],
                          axis=1).reshape(NW, NCHUNK, CHUNK)
    col = jnp.concatenate([ei[1].reshape(NW, EPW), fill_c],
                          axis=1).reshape(NW, NCHUNK, CHUNK)
    colp = jnp.concatenate(
        [ei[1], jnp.full((PAD,), N, jnp.int32)]).reshape(NW, EPW_P)
    zeros_n = jnp.zeros((NP, 1), jnp.float32)
    z64 = jnp.zeros((SLAB, 64), jnp.float32)
    hists = _degree_kernel(colp, zeros_n)
    *xss, dinv = _prep(x, hists)
    ps = _prop_l1(*xss, row, col, z64)
    z = _mm(ps, xss, dinv, W1, b1[None, :], W2)
    (q,) = _prop_l2(z, row, col, z64)
    out = _final(q, z, dinv, b2[None, :])
    return (out, edge_index)


# speedup vs baseline: 1.0411x; 1.0041x over previous
import functools

import jax
import jax.numpy as jnp
from jax import lax
from jax.experimental import pallas as pl
from jax.experimental.pallas import tpu as pltpu
from jax.experimental.pallas import tpu_sc as plsc

N = 10000
E = 160000
NC = 2
NS = 16
NW = NC * NS
EPW = E // NW
CHUNK = 128
EPT = 5120
GROUP = 4
NCHUNK = EPT // CHUNK
NGROUP = NCHUNK // GROUP
NP = 10240
SLAB = NP // NS
EPW_P = 5008
PAD = NW * EPW_P - E
RED = 128
BLK = 1000

_MESH = plsc.VectorSubcoreMesh(core_axis_name="c", subcore_axis_name="s",
                               num_cores=NC, num_subcores=NS)
_SC_PARAMS = pltpu.CompilerParams(needs_layout_passes=False,
                                  use_tc_tiling_on_sc=False)


@functools.partial(
    pl.kernel,
    out_type=jax.ShapeDtypeStruct((NW, NP, 1), jnp.float32),
    mesh=_MESH,
    compiler_params=_SC_PARAMS,
    scratch_types=[
        pltpu.VMEM((EPW_P,), jnp.int32),
        pltpu.VMEM((NP, 1), jnp.float32),
    ],
)
def _degree_kernel(col_hbm, zeros_hbm, hist_out, colv, hist):
    cid = lax.axis_index("c")
    sid = lax.axis_index("s")
    wid = cid * NS + sid
    pltpu.sync_copy(col_hbm.at[wid], colv)
    pltpu.sync_copy(zeros_hbm, hist)
    ones = jnp.full((16,), 1.0, jnp.float32)
    zcol = jnp.zeros((16,), jnp.int32)

    def body(j, carry):
        idx = colv[pl.ds(j * 16, 16)]
        plsc.addupdate_scatter(hist, [idx, zcol], ones)
        return carry

    lax.fori_loop(0, EPW_P // 16, body, 0)
    pltpu.sync_copy(hist, hist_out.at[wid])


def _make_propagate(ntab, pipelined):
    nbuf = 2 * GROUP if pipelined else GROUP
    scratch = (
        [pltpu.VMEM((NCHUNK, CHUNK), jnp.int32) for _ in range(2)]
        + [pltpu.VMEM((CHUNK, 64), jnp.float32) for _ in range(nbuf)]
        + [pltpu.VMEM_SHARED((NP, 64), jnp.float32),
           pltpu.SemaphoreType.DMA, pltpu.SemaphoreType.DMA]
    )
    out_type = [jax.ShapeDtypeStruct((NC, NP, 64), jnp.float32)
                for _ in range(ntab)]

    @functools.partial(pl.kernel, out_type=out_type, mesh=_MESH,
                       compiler_params=_SC_PARAMS, scratch_types=scratch)
    def prop(*refs):
        tabs = refs[:ntab]
        row_hbm, col_hbm, zeros_hbm = refs[ntab:ntab + 3]
        outs = refs[ntab + 3:2 * ntab + 3]
        rowbuf, colbuf = refs[2 * ntab + 3:2 * ntab + 5]
        allbufs = refs[2 * ntab + 5:2 * ntab + 5 + nbuf]
        bufs = (allbufs[:GROUP], allbufs[GROUP:])
        acc = refs[-3]
        sems = (refs[-2], refs[-1])
        cid = lax.axis_index("c")
        sid = lax.axis_index("s")
        wid = cid * NS + sid
        pltpu.sync_copy(row_hbm.at[wid], rowbuf)
        pltpu.sync_copy(col_hbm.at[wid], colbuf)
        slab = pl.ds(sid * SLAB, SLAB)

        def fire(t, g, s):
            return [pltpu.async_copy(
                tabs[t].at[rowbuf.at[g * GROUP + b]], bufs[s][b], sems[s])
                for b in range(GROUP)]

        def scat(g, s):
            for b in range(GROUP):
                pltpu.sync_copy(bufs[s][b], acc.at[colbuf.at[g * GROUP + b]],
                                add=True)

        for t in range(ntab):
            pltpu.sync_copy(zeros_hbm, acc.at[slab])
            plsc.subcore_barrier()
            if pipelined:
                def body(it, carry, t=t):
                    g0 = it * 2
                    da = fire(t, g0, 0)
                    db = fire(t, g0 + 1, 1)
                    for d in da:
                        d.wait()
                    scat(g0, 0)
                    for d in db:
                        d.wait()
                    scat(g0 + 1, 1)
                    return carry

                lax.fori_loop(0, NGROUP // 2, body, 0)
            else:
                def body(g, carry, t=t):
                    descs = fire(t, g, 0)
                    for d in descs:
                        d.wait()
                    scat(g, 0)
                    return carry

                lax.fori_loop(0, NGROUP, body, 0)
            plsc.subcore_barrier()
            pltpu.sync_copy(acc.at[slab], outs[t].at[cid, slab])

    return prop


_prop_l1 = _make_propagate(4, pipelined=True)
_prop_l2 = _make_propagate(1, pipelined=True)


def _prep(x, hists):
    def body(x_ref, h_ref, xs0_ref, xs1_ref, xs2_ref, xs3_ref, dinv_ref):
        deg = jnp.sum(h_ref[...], axis=0) + 1.0
        dinv = lax.rsqrt(deg)
        xs = x_ref[...] * dinv
        xs0_ref[...] = xs[:, 0:64]
        xs1_ref[...] = xs[:, 64:128]
        xs2_ref[...] = xs[:, 128:192]
        xs3_ref[...] = xs[:, 192:256]
        dinv_ref[...] = dinv

    return pl.pallas_call(
        body,
        grid=(N // BLK,),
        in_specs=[
            pl.BlockSpec((BLK, 256), lambda i: (i, 0)),
            pl.BlockSpec((NW, BLK, 1), lambda i: (0, i, 0)),
        ],
        out_specs=[pl.BlockSpec((BLK, 64), lambda i: (i, 0))] * 4
        + [pl.BlockSpec((BLK, 1), lambda i: (i, 0))],
        out_shape=[jax.ShapeDtypeStruct((N, 64), jnp.float32)] * 4
        + [jax.ShapeDtypeStruct((N, 1), jnp.float32)],
    )(x, hists)


def _mm(ps, xss, dinv, W1, b1, W2):
    def body(p0_ref, p1_ref, p2_ref, p3_ref, xs0_ref, xs1_ref, xs2_ref,
             xs3_ref, dinv_ref, W1_ref, b1_ref, W2_ref, z_ref):
        dv = dinv_ref[...]
        p_refs = (p0_ref, p1_ref, p2_ref, p3_ref)
        xs_refs = (xs0_ref, xs1_ref, xs2_ref, xs3_ref)
        h = b1_ref[...]
        for k in range(4):
            u = (p_refs[k][0] + p_refs[k][1] + xs_refs[k][...]) * dv
            h = h + jnp.dot(u, W1_ref[64 * k:64 * (k + 1), :],
                            preferred_element_type=jnp.float32)
        hs = jnp.maximum(h, 0.0) * dv
        z_ref[...] = jnp.dot(hs, W2_ref[...], preferred_element_type=jnp.float32)

    return pl.pallas_call(
        body,
        grid=(N // BLK,),
        in_specs=[pl.BlockSpec((NC, BLK, 64), lambda i: (0, i, 0))] * 4
        + [pl.BlockSpec((BLK, 64), lambda i: (i, 0))] * 4
        + [
            pl.BlockSpec((BLK, 1), lambda i: (i, 0)),
            pl.BlockSpec((256, 512), lambda i: (0, 0)),
            pl.BlockSpec((1, 512), lambda i: (0, 0)),
            pl.BlockSpec((512, 64), lambda i: (0, 0)),
        ],
        out_specs=pl.BlockSpec((BLK, 64), lambda i: (i, 0)),
        out_shape=jax.ShapeDtypeStruct((N, 64), jnp.float32),
    )(*ps, *xss, dinv, W1, b1, W2)


def _final(q, z, dinv, b2):
    def body(q_ref, z_ref, dinv_ref, b2_ref, out_ref):
        agg = q_ref[0] + q_ref[1] + z_ref[...]
        out_ref[...] = agg * dinv_ref[...] + b2_ref[...]

    return pl.pallas_call(
        body,
        grid=(N // BLK,),
        in_specs=[
            pl.BlockSpec((NC, BLK, 64), lambda i: (0, i, 0)),
            pl.BlockSpec((BLK, 64), lambda i: (i, 0)),
            pl.BlockSpec((BLK, 1), lambda i: (i, 0)),
            pl.BlockSpec((1, 64), lambda i: (0, 0)),
        ],
        out_specs=pl.BlockSpec((BLK, 64), lambda i: (i, 0)),
        out_shape=jax.ShapeDtypeStruct((N, 64), jnp.float32),
    )(q, z, dinv, b2)


def kernel(x, edge_index, W1, b1, W2, b2):
    ei = edge_index.astype(jnp.int32)
    npad = NW * (EPT - EPW)
    fill_r = (jnp.arange(npad, dtype=jnp.int32) % N).reshape(NW, EPT - EPW)
    fill_c = (N + jnp.arange(npad, dtype=jnp.int32) % (NP - N)).reshape(
        NW, EPT - EPW)
    row = jnp.concatenate([ei[0].reshape(NW, EPW), fill_r],
                          axis=1).reshape(NW, NCHUNK, CHUNK)
    col = jnp.concatenate([ei[1].reshape(NW, EPW), fill_c],
                          axis=1).reshape(NW, NCHUNK, CHUNK)
    colp = jnp.concatenate(
        [ei[1], jnp.full((PAD,), N, jnp.int32)]).reshape(NW, EPW_P)
    zeros_n = jnp.zeros((NP, 1), jnp.float32)
    z64 = jnp.zeros((SLAB, 64), jnp.float32)
    hists = _degree_kernel(colp, zeros_n)
    *xss, dinv = _prep(x, hists)
    ps = _prop_l1(*xss, row, col, z64)
    z = _mm(ps, xss, dinv, W1, b1[None, :], W2)
    (q,) = _prop_l2(z, row, col, z64)
    out = _final(q, z, dinv, b2[None, :])
    return (out, edge_index)
